# Initial kernel scaffold; baseline (speedup 1.0000x reference)
import functools

import jax
import jax.numpy as jnp
from jax import lax
from jax.experimental import pallas as pl
from jax.experimental.pallas import tpu as pltpu
from jax.experimental.pallas import tpu_sc as plsc

D = 64
L = 16
NC = 2
NS = 16
NW = NC * NS
GROUP = 128
NBUF = 4


def _build(n_total):
    assert n_total % (NW * GROUP) == 0
    n_per_w = n_total // NW
    n_groups = n_per_w // GROUP
    assert n_groups % NBUF == 0

    mesh = plsc.VectorSubcoreMesh(core_axis_name="c", subcore_axis_name="s")

    @functools.partial(
        pl.kernel,
        out_type=jax.ShapeDtypeStruct((n_total, D), jnp.float32),
        mesh=mesh,
        scratch_types=[
            pltpu.VMEM((n_groups, GROUP), jnp.int32),
            pltpu.VMEM((n_groups, GROUP), jnp.int32),
            pltpu.VMEM((NBUF, GROUP, D), jnp.float32),
            pltpu.VMEM((NBUF, GROUP, D), jnp.float32),
            pltpu.SemaphoreType.DMA((NBUF,)),
        ],
    )
    def kern(tok_idx_hbm, diac_idx_hbm, tok_tab_hbm, diac_tab_hbm, out_hbm,
             it_v, id_v, tr_v, dr_v, sems):
        wid = lax.axis_index("s") * NC + lax.axis_index("c")
        row0 = wid * n_groups
        base = wid * n_per_w

        pltpu.sync_copy(tok_idx_hbm.at[pl.ds(row0, n_groups)], it_v)
        pltpu.sync_copy(diac_idx_hbm.at[pl.ds(row0, n_groups)], id_v)

        @pl.loop(0, n_groups, step=NBUF)
        def _(g0):
            descs = []
            for b in range(NBUF):
                g = g0 + b
                dt = pltpu.async_copy(
                    tok_tab_hbm.at[it_v.at[g]], tr_v.at[b], sems.at[b])
                dd = pltpu.async_copy(
                    diac_tab_hbm.at[id_v.at[g]], dr_v.at[b], sems.at[b])
                descs.append((dt, dd))
            for b in range(NBUF):
                descs[b][0].wait()
                descs[b][1].wait()

                @pl.loop(0, GROUP, unroll=4)
                def _(i):
                    for j in range(D // L):
                        s = pl.ds(j * L, L)
                        tr_v[b, i, s] = tr_v[b, i, s] + dr_v[b, i, s]

                g = g0 + b
                pltpu.sync_copy(
                    tr_v.at[b], out_hbm.at[pl.ds(base + g * GROUP, GROUP)])

    return kern


_kern = _build(4096 * 200)


def kernel(token_inputs, diac_inputs, token_table, diac_table):
    B, S = token_inputs.shape
    n = B * S
    tok_idx = token_inputs.reshape(n // GROUP, GROUP)
    diac_idx = diac_inputs.reshape(n // GROUP, GROUP)
    out = _kern(tok_idx, diac_idx, token_table, diac_table)
    return out.reshape(B, S, D)


# speedup vs baseline: 5.2307x; 5.2307x over previous
import functools

import jax
import jax.numpy as jnp
from jax import lax
from jax.experimental import pallas as pl
from jax.experimental.pallas import tpu as pltpu
from jax.experimental.pallas import tpu_sc as plsc

D = 64
L = 16
NC = 2
NS = 16
NW = NC * NS
GROUP = 128
NBUF = 4


def _build(n_total):
    assert n_total % (NW * GROUP) == 0
    n_per_w = n_total // NW
    n_groups = n_per_w // GROUP
    assert n_groups % NBUF == 0

    mesh = plsc.VectorSubcoreMesh(core_axis_name="c", subcore_axis_name="s")

    @functools.partial(
        pl.kernel,
        out_type=jax.ShapeDtypeStruct((n_total, D), jnp.float32),
        mesh=mesh,
        scratch_types=[
            pltpu.VMEM((n_groups, GROUP), jnp.int32),
            pltpu.VMEM((n_groups, GROUP), jnp.int32),
            pltpu.VMEM((NBUF, GROUP, D), jnp.float32),
            pltpu.VMEM((NBUF, GROUP, D), jnp.float32),
            pltpu.SemaphoreType.DMA((NBUF,)),
        ],
        compiler_params=pltpu.CompilerParams(use_tc_tiling_on_sc=False),
    )
    def kern(tok_idx_hbm, diac_idx_hbm, tok_tab_hbm, diac_tab_hbm, out_hbm,
             it_v, id_v, tr_v, dr_v, sems):
        wid = lax.axis_index("s") * NC + lax.axis_index("c")
        row0 = wid * n_groups
        base = wid * n_per_w

        pltpu.sync_copy(tok_idx_hbm.at[pl.ds(row0, n_groups)], it_v)
        pltpu.sync_copy(diac_idx_hbm.at[pl.ds(row0, n_groups)], id_v)

        @pl.loop(0, n_groups, step=NBUF)
        def _(g0):
            descs = []
            for b in range(NBUF):
                g = g0 + b
                dt = pltpu.async_copy(
                    tok_tab_hbm.at[it_v.at[g]], tr_v.at[b], sems.at[b])
                dd = pltpu.async_copy(
                    diac_tab_hbm.at[id_v.at[g]], dr_v.at[b], sems.at[b])
                descs.append((dt, dd))
            for b in range(NBUF):
                descs[b][0].wait()
                descs[b][1].wait()

                @pl.loop(0, GROUP, unroll=4)
                def _(i):
                    for j in range(D // L):
                        s = pl.ds(j * L, L)
                        tr_v[b, i, s] = tr_v[b, i, s] + dr_v[b, i, s]

                g = g0 + b
                pltpu.sync_copy(
                    tr_v.at[b], out_hbm.at[pl.ds(base + g * GROUP, GROUP)])

    return kern


_kern = _build(4096 * 200)


def kernel(token_inputs, diac_inputs, token_table, diac_table):
    B, S = token_inputs.shape
    n = B * S
    tok_idx = token_inputs.reshape(n // GROUP, GROUP)
    diac_idx = diac_inputs.reshape(n // GROUP, GROUP)
    out = _kern(tok_idx, diac_idx, token_table, diac_table)
    return out.reshape(B, S, D)


# speedup vs baseline: 5.2915x; 1.0116x over previous
import functools

import jax
import jax.numpy as jnp
from jax import lax
from jax.experimental import pallas as pl
from jax.experimental.pallas import tpu as pltpu
from jax.experimental.pallas import tpu_sc as plsc

D = 64
L = 16
NC = 2
NS = 16
NW = NC * NS
GROUP = 128
NBUF = 2


def _build(n_total):
    assert n_total % (NW * GROUP) == 0
    n_per_w = n_total // NW
    n_groups = n_per_w // GROUP
    assert n_groups % NBUF == 0

    mesh = plsc.VectorSubcoreMesh(core_axis_name="c", subcore_axis_name="s")

    @functools.partial(
        pl.kernel,
        out_type=jax.ShapeDtypeStruct((n_total, D), jnp.float32),
        mesh=mesh,
        scratch_types=[
            pltpu.VMEM((n_groups, GROUP), jnp.int32),
            pltpu.VMEM((n_groups, GROUP), jnp.int32),
            pltpu.VMEM((NBUF, GROUP, D), jnp.float32),
            pltpu.VMEM((NBUF, GROUP, D), jnp.float32),
            pltpu.VMEM((NBUF, GROUP, D), jnp.float32),
            pltpu.SemaphoreType.DMA((NBUF,)),
            pltpu.SemaphoreType.DMA((NBUF,)),
        ],
        compiler_params=pltpu.CompilerParams(use_tc_tiling_on_sc=False),
    )
    def kern(tok_idx_hbm, diac_idx_hbm, tok_tab_hbm, diac_tab_hbm, out_hbm,
             it_v, id_v, tr_v, dr_v, ob_v, gsems, wsems):
        wid = lax.axis_index("s") * NC + lax.axis_index("c")
        row0 = wid * n_groups
        base = wid * n_per_w

        pltpu.sync_copy(tok_idx_hbm.at[pl.ds(row0, n_groups)], it_v)
        pltpu.sync_copy(diac_idx_hbm.at[pl.ds(row0, n_groups)], id_v)

        def issue_gathers(g, b):
            pltpu.async_copy(tok_tab_hbm.at[it_v.at[g]], tr_v.at[b], gsems.at[b])
            pltpu.async_copy(diac_tab_hbm.at[id_v.at[g]], dr_v.at[b], gsems.at[b])

        def wait_gathers(g, b):
            pltpu.make_async_copy(
                tok_tab_hbm.at[it_v.at[g]], tr_v.at[b], gsems.at[b]).wait()
            pltpu.make_async_copy(
                diac_tab_hbm.at[id_v.at[g]], dr_v.at[b], gsems.at[b]).wait()

        def out_slice(g):
            return out_hbm.at[pl.ds(base + g * GROUP, GROUP)]

        def add_group(b):
            @pl.loop(0, GROUP, unroll=4)
            def _(i):
                for j in range(D // L):
                    s = pl.ds(j * L, L)
                    ob_v[b, i, s] = tr_v[b, i, s] + dr_v[b, i, s]

        for b in range(NBUF):
            issue_gathers(b, b)

        @pl.loop(0, n_groups - NBUF, step=NBUF)
        def _(g0):
            for b in range(NBUF):
                g = g0 + b
                wait_gathers(g, b)

                @pl.when(g0 >= NBUF)
                def _():
                    pltpu.make_async_copy(
                        ob_v.at[b], out_slice(g - NBUF), wsems.at[b]).wait()

                add_group(b)
                pltpu.async_copy(ob_v.at[b], out_slice(g), wsems.at[b])
                issue_gathers(g + NBUF, b)

        for b in range(NBUF):
            g = n_groups - NBUF + b
            wait_gathers(g, b)
            pltpu.make_async_copy(
                ob_v.at[b], out_slice(g - NBUF), wsems.at[b]).wait()
            add_group(b)
            pltpu.async_copy(ob_v.at[b], out_slice(g), wsems.at[b])
        for b in range(NBUF):
            g = n_groups - NBUF + b
            pltpu.make_async_copy(
                ob_v.at[b], out_slice(g), wsems.at[b]).wait()

    return kern


_kern = _build(4096 * 200)


def kernel(token_inputs, diac_inputs, token_table, diac_table):
    B, S = token_inputs.shape
    n = B * S
    tok_idx = token_inputs.reshape(n // GROUP, GROUP)
    diac_idx = diac_inputs.reshape(n // GROUP, GROUP)
    out = _kern(tok_idx, diac_idx, token_table, diac_table)
    return out.reshape(B, S, D)
